# baseline (speedup 1.0000x reference)
import jax
import jax.numpy as jnp
from jax import lax
from jax.experimental import pallas as pl
from jax.experimental.pallas import tpu as pltpu
from jax.experimental.pallas import tpu_sc as plsc

_B = 8
_D = 512
_TOTAL = 15488
_NPOS = 8192
_ALPHA = float(_D) ** 0.5
_L = 16
_CHUNK = 16
_NCHUNKS = _TOTAL // _CHUNK
_NW = 32
_NMAX = -(-_NCHUNKS // _NW)
_R = 4


def _body(meta_hbm, emb_hbm, pos_hbm, out_hbm, meta_v, *rest):
  idxs = rest[0:_R]
  embs = rest[_R:2 * _R]
  poss = rest[2 * _R:3 * _R]
  outs = rest[3 * _R:4 * _R]
  ses = rest[4 * _R:5 * _R]
  sps = rest[5 * _R:6 * _R]
  sos = rest[6 * _R:7 * _R]

  cid = lax.axis_index("c")
  sid = lax.axis_index("s")
  wid = sid * 2 + cid

  pltpu.sync_copy(meta_hbm, meta_v)
  off = [meta_v[b, :] for b in range(_B)]
  high = [meta_v[_B + b, :] for b in range(_B)]
  lanes = lax.iota(jnp.int32, _L)

  ge8 = (wid + (_NW - 8)) // _NW
  nloc = 31 - ge8
  start_w = 31 * wid - ge8 * (wid - 8)

  def compute_idx(base, idx_ref):
    t = base + lanes
    off_s = off[0]
    high_s = high[0]
    for s in range(1, _B):
      m = t >= off[s]
      off_s = jnp.where(m, off[s], off_s)
      high_s = jnp.where(m, high[s], high_s)
    p = jnp.minimum(t - off_s, high_s)
    p = jnp.maximum(jnp.minimum(p, _NPOS - 1), 0)
    idx_ref[...] = p

  for b in range(_R):
    base = (start_w + b) * _CHUNK
    compute_idx(base, idxs[b])
    pltpu.make_async_copy(pos_hbm.at[idxs[b]], poss[b], sps[b]).start()
    pltpu.make_async_copy(emb_hbm.at[pl.ds(base, _CHUNK)], embs[b], ses[b]).start()

  def quad_body(i, carry):
    for b in range(_R):
      slot = _R * i + b

      @pl.when(slot < nloc)
      def _do(slot=slot, b=b):
        base = (start_w + slot) * _CHUNK
        pltpu.make_async_copy(
            emb_hbm.at[pl.ds(base, _CHUNK)], embs[b], ses[b]).wait()
        pltpu.make_async_copy(pos_hbm.at[idxs[b]], poss[b], sps[b]).wait()

        @pl.when(slot >= _R)
        def _wait_prev_store():
          pltpu.make_async_copy(
              outs[b], out_hbm.at[pl.ds(0, _CHUNK)], sos[b]).wait()

        def row_body(r2, c2):
          for dr in range(2):
            r = r2 * 2 + dr
            for k in range(_D // _L):
              sl = pl.ds(k * _L, _L)
              outs[b][r, sl] = embs[b][r, sl] * _ALPHA + poss[b][r, sl]
          return c2
        lax.fori_loop(0, _CHUNK // 2, row_body, 0)

        pltpu.make_async_copy(
            outs[b], out_hbm.at[pl.ds(base, _CHUNK)], sos[b]).start()

        @pl.when(slot + _R < nloc)
        def _prefetch(slot=slot, b=b):
          base2 = base + _R * _CHUNK
          compute_idx(base2, idxs[b])
          pltpu.make_async_copy(pos_hbm.at[idxs[b]], poss[b], sps[b]).start()
          pltpu.make_async_copy(
              emb_hbm.at[pl.ds(base2, _CHUNK)], embs[b], ses[b]).start()
    return carry

  lax.fori_loop(0, (_NMAX + _R - 1) // _R, quad_body, 0)

  for b in range(_R):
    pltpu.make_async_copy(outs[b], out_hbm.at[pl.ds(0, _CHUNK)], sos[b]).wait()


def kernel(max_seq_len, seq_lengths, seq_offsets, seq_embeddings,
           num_targets, pos_weight):
  high = jnp.minimum(seq_lengths - num_targets, _NPOS - 1).astype(jnp.int32)
  meta = jnp.concatenate([
      jnp.broadcast_to(seq_offsets[:_B, None].astype(jnp.int32), (_B, _L)),
      jnp.broadcast_to(high[:, None], (_B, _L)),
  ], axis=0)

  scratch = [pltpu.VMEM((2 * _B, _L), jnp.int32)]
  scratch += [pltpu.VMEM((_CHUNK,), jnp.int32) for _ in range(_R)]
  scratch += [pltpu.VMEM((_CHUNK, _D), jnp.float32) for _ in range(3 * _R)]
  scratch += [pltpu.SemaphoreType.DMA for _ in range(3 * _R)]

  f = pl.kernel(
      _body,
      out_type=jax.ShapeDtypeStruct((_TOTAL, _D), jnp.float32),
      mesh=plsc.VectorSubcoreMesh(core_axis_name="c", subcore_axis_name="s"),
      scratch_types=scratch,
  )
  return f(meta, seq_embeddings, pos_weight)


# speedup vs baseline: 1.1627x; 1.1627x over previous
import jax
import jax.numpy as jnp
from jax import lax
from jax.experimental import pallas as pl
from jax.experimental.pallas import tpu as pltpu
from jax.experimental.pallas import tpu_sc as plsc

_B = 8
_D = 512
_TOTAL = 15488
_NPOS = 8192
_ALPHA = float(_D) ** 0.5
_L = 16
_CHUNK = 16
_NCHUNKS = _TOTAL // _CHUNK
_NW = 32
_NMAX = -(-_NCHUNKS // _NW)
_R = 4


def _body(meta_hbm, emb_hbm, pos_hbm, out_hbm, meta_v, *rest):
  idxs = rest[0:_R]
  embs = rest[_R:2 * _R]
  poss = rest[2 * _R:3 * _R]
  outs = rest[3 * _R:4 * _R]
  ses = rest[4 * _R:5 * _R]
  sps = rest[5 * _R:6 * _R]
  sos = rest[6 * _R:7 * _R]

  cid = lax.axis_index("c")
  sid = lax.axis_index("s")
  wid = sid * 2 + cid

  pltpu.sync_copy(meta_hbm, meta_v)
  off = [meta_v[b, :] for b in range(_B)]
  high = [meta_v[_B + b, :] for b in range(_B)]
  lanes = lax.iota(jnp.int32, _L)

  ge8 = (wid + (_NW - 8)) // _NW
  nloc = 31 - ge8
  start_w = 31 * wid - ge8 * (wid - 8)

  def compute_idx(base, idx_ref):
    t = base + lanes
    off_s = off[0]
    high_s = high[0]
    for s in range(1, _B):
      m = t >= off[s]
      off_s = jnp.where(m, off[s], off_s)
      high_s = jnp.where(m, high[s], high_s)
    p = jnp.minimum(t - off_s, high_s)
    p = jnp.maximum(jnp.minimum(p, _NPOS - 1), 0)
    idx_ref[...] = p

  for b in range(_R):
    base = (start_w + b) * _CHUNK
    compute_idx(base, idxs[b])
    pltpu.make_async_copy(pos_hbm.at[idxs[b]], poss[b], sps[b]).start()
    pltpu.make_async_copy(emb_hbm.at[pl.ds(base, _CHUNK)], embs[b], ses[b]).start()

  def quad_body(i, carry):
    for b in range(_R):
      slot = _R * i + b

      @pl.when(slot < nloc)
      def _do(slot=slot, b=b):
        base = (start_w + slot) * _CHUNK
        pltpu.make_async_copy(
            emb_hbm.at[pl.ds(base, _CHUNK)], embs[b], ses[b]).wait()
        pltpu.make_async_copy(pos_hbm.at[idxs[b]], poss[b], sps[b]).wait()

        @pl.when(slot >= _R)
        def _wait_prev_store():
          pltpu.make_async_copy(
              outs[b], out_hbm.at[pl.ds(0, _CHUNK)], sos[b]).wait()

        def row_body(r, c2):
          for k in range(_D // _L):
            sl = pl.ds(k * _L, _L)
            outs[b][r, sl] = embs[b][r, sl] * _ALPHA + poss[b][r, sl]
          return c2
        lax.fori_loop(0, _CHUNK, row_body, 0)

        pltpu.make_async_copy(
            outs[b], out_hbm.at[pl.ds(base, _CHUNK)], sos[b]).start()

        @pl.when(slot + _R < nloc)
        def _prefetch(slot=slot, b=b):
          base2 = base + _R * _CHUNK
          compute_idx(base2, idxs[b])
          pltpu.make_async_copy(pos_hbm.at[idxs[b]], poss[b], sps[b]).start()
          pltpu.make_async_copy(
              emb_hbm.at[pl.ds(base2, _CHUNK)], embs[b], ses[b]).start()
    return carry

  lax.fori_loop(0, (_NMAX + _R - 1) // _R, quad_body, 0)

  for b in range(_R):
    pltpu.make_async_copy(outs[b], out_hbm.at[pl.ds(0, _CHUNK)], sos[b]).wait()


def kernel(max_seq_len, seq_lengths, seq_offsets, seq_embeddings,
           num_targets, pos_weight):
  high = jnp.minimum(seq_lengths - num_targets, _NPOS - 1).astype(jnp.int32)
  meta = jnp.concatenate([
      jnp.broadcast_to(seq_offsets[:_B, None].astype(jnp.int32), (_B, _L)),
      jnp.broadcast_to(high[:, None], (_B, _L)),
  ], axis=0)

  scratch = [pltpu.VMEM((2 * _B, _L), jnp.int32)]
  scratch += [pltpu.VMEM((_CHUNK,), jnp.int32) for _ in range(_R)]
  scratch += [pltpu.VMEM((_CHUNK, _D), jnp.float32) for _ in range(3 * _R)]
  scratch += [pltpu.SemaphoreType.DMA for _ in range(3 * _R)]

  f = pl.kernel(
      _body,
      out_type=jax.ShapeDtypeStruct((_TOTAL, _D), jnp.float32),
      mesh=plsc.VectorSubcoreMesh(core_axis_name="c", subcore_axis_name="s"),
      scratch_types=scratch,
  )
  return f(meta, seq_embeddings, pos_weight)
